# baseline (speedup 1.0000x reference)
import functools

import jax
import jax.numpy as jnp
from jax import lax
from jax.experimental import pallas as pl
from jax.experimental.pallas import tpu as pltpu


def _top2_body(s_ref, vals_ref, idxs_ref):
    s = s_ref[...]
    c = s.shape[-1]
    iota = lax.broadcasted_iota(jnp.int32, s.shape, 1)
    m1 = jnp.max(s, axis=-1, keepdims=True)
    i1 = jnp.min(jnp.where(s == m1, iota, c), axis=-1, keepdims=True)
    s2 = jnp.where(iota == i1, -jnp.inf, s)
    m2 = jnp.max(s2, axis=-1, keepdims=True)
    i2 = jnp.min(jnp.where(s2 == m2, iota, c), axis=-1, keepdims=True)
    vals_ref[:, 0:1] = m1
    vals_ref[:, 1:2] = m2
    idxs_ref[:, 0:1] = i1
    idxs_ref[:, 1:2] = i2


def _run_top2(scores, tn):
    n, c = scores.shape
    return pl.pallas_call(
        _top2_body,
        grid=(n // tn,),
        in_specs=[pl.BlockSpec((tn, c), lambda t: (t, 0))],
        out_specs=[
            pl.BlockSpec((tn, 2), lambda t: (t, 0)),
            pl.BlockSpec((tn, 2), lambda t: (t, 0)),
        ],
        out_shape=[
            jax.ShapeDtypeStruct((n, 2), jnp.float32),
            jax.ShapeDtypeStruct((n, 2), jnp.int32),
        ],
    )(scores)


def _sparse_dense_body(nk, x_ref, w_ref, vo_ref, io_ref, vi_ref, ii_ref,
                       bias_ref, out_ref, pacc, bacc):
    kb = pl.program_id(1)
    kw = x_ref.shape[1]

    @pl.when(kb == 0)
    def _():
        pacc[...] = jnp.zeros_like(pacc)
        bacc[...] = jnp.zeros_like(bacc)

    x_b = x_ref[...]
    ci = kb * kw + lax.broadcasted_iota(jnp.int32, x_b.shape, 1)
    sin = (jnp.where(ci == ii_ref[:, 0:1], vi_ref[:, 0:1], 0.0)
           + jnp.where(ci == ii_ref[:, 1:2], vi_ref[:, 1:2], 0.0))
    w_b = w_ref[...]
    xs = (x_b.astype(jnp.float32) * sin).astype(jnp.bfloat16)
    pacc[...] += jnp.dot(x_b, w_b, preferred_element_type=jnp.float32)
    bacc[...] += jnp.dot(xs, w_b, preferred_element_type=jnp.float32)

    @pl.when(kb == nk - 1)
    def _():
        p = pacc[...]
        co = lax.broadcasted_iota(jnp.int32, p.shape, 1)
        sout = (jnp.where(co == io_ref[:, 0:1], vo_ref[:, 0:1], 0.0)
                + jnp.where(co == io_ref[:, 1:2], vo_ref[:, 1:2], 0.0))
        out_ref[...] = p * sout + bacc[...] + bias_ref[...]


def _run_sparse_dense(xf, w, vo, io, vi, ii, bias2d, tb, kbw):
    n, c1 = xf.shape
    c2 = w.shape[1]
    nk = c1 // kbw
    grid = (n // tb, nk)
    return pl.pallas_call(
        functools.partial(_sparse_dense_body, nk),
        grid=grid,
        in_specs=[
            pl.BlockSpec((tb, kbw), lambda t, k: (t, k)),
            pl.BlockSpec((kbw, c2), lambda t, k: (k, 0)),
            pl.BlockSpec((tb, 2), lambda t, k: (t, 0)),
            pl.BlockSpec((tb, 2), lambda t, k: (t, 0)),
            pl.BlockSpec((tb, 2), lambda t, k: (t, 0)),
            pl.BlockSpec((tb, 2), lambda t, k: (t, 0)),
            pl.BlockSpec((1, c2), lambda t, k: (0, 0)),
        ],
        out_specs=pl.BlockSpec((tb, c2), lambda t, k: (t, 0)),
        out_shape=jax.ShapeDtypeStruct((n, c2), jnp.float32),
        scratch_shapes=[
            pltpu.VMEM((tb, c2), jnp.float32),
            pltpu.VMEM((tb, c2), jnp.float32),
        ],
        compiler_params=pltpu.CompilerParams(
            dimension_semantics=("arbitrary", "arbitrary")),
    )(xf, w, vo, io, vi, ii, bias2d)


def kernel(x, weight, bias, so_cw, so_cb, so_bg, so_bb, so_bm, so_bv, so_ew,
           so_eb, si_cw, si_cb, si_bg, si_bb, si_bm, si_bv, si_ew, si_eb):
    b, l, c1 = x.shape
    c2 = weight.shape[1]
    n = b * l
    g = so_cw.shape[0]
    xf = x.reshape(n, c1)

    def gb(cw, cb, bg, bb, bm, bv, ew, eb):
        xr = xf.reshape(n, g, 4)
        comp = jnp.einsum('ngi,gi->ng', xr, cw) + cb
        bnv = (comp - bm) / jnp.sqrt(bv + 1e-5) * bg + bb
        act = jax.nn.gelu(bnv, approximate=False)
        return act @ ew.T + eb

    sc_o = jax.nn.softmax(
        gb(so_cw, so_cb, so_bg, so_bb, so_bm, so_bv, so_ew,
           so_eb).reshape(b, l, c2), axis=-1)
    sc_i = jax.nn.softmax(
        gb(si_cw, si_cb, si_bg, si_bb, si_bm, si_bv, si_ew,
           si_eb).reshape(b, l, c1), axis=-1)

    tn = min(256, n)
    vo, io = _run_top2(sc_o.reshape(n, c2), tn)
    vi, ii = _run_top2(sc_i.reshape(n, c1), tn)

    tb = min(512, n)
    kbw = min(512, c1)
    out = _run_sparse_dense(xf.astype(jnp.bfloat16),
                            weight.astype(jnp.bfloat16), vo, io, vi, ii,
                            bias.reshape(1, c2), tb, kbw)
    return out.reshape(b, l, c2)


# speedup vs baseline: 1.0639x; 1.0639x over previous
import functools

import jax
import jax.numpy as jnp
from jax import lax
from jax.experimental import pallas as pl
from jax.experimental.pallas import tpu as pltpu


def _top2_body(s_ref, vals_ref, idxs_ref):
    s = s_ref[...]
    c = s.shape[-1]
    iota = lax.broadcasted_iota(jnp.int32, s.shape, 1)
    m1 = jnp.max(s, axis=-1, keepdims=True)
    i1 = jnp.min(jnp.where(s == m1, iota, c), axis=-1, keepdims=True)
    s2 = jnp.where(iota == i1, -jnp.inf, s)
    m2 = jnp.max(s2, axis=-1, keepdims=True)
    i2 = jnp.min(jnp.where(s2 == m2, iota, c), axis=-1, keepdims=True)
    vals_ref[:, 0:1] = m1
    vals_ref[:, 1:2] = m2
    idxs_ref[:, 0:1] = i1
    idxs_ref[:, 1:2] = i2


def _run_top2(scores, tn):
    n, c = scores.shape
    return pl.pallas_call(
        _top2_body,
        grid=(n // tn,),
        in_specs=[pl.BlockSpec((tn, c), lambda t: (t, 0))],
        out_specs=[
            pl.BlockSpec((tn, 2), lambda t: (t, 0)),
            pl.BlockSpec((tn, 2), lambda t: (t, 0)),
        ],
        out_shape=[
            jax.ShapeDtypeStruct((n, 2), jnp.float32),
            jax.ShapeDtypeStruct((n, 2), jnp.int32),
        ],
    )(scores)


def _sparse_dense_body(x_ref, w_ref, vo_ref, io_ref, vi_ref, ii_ref,
                       bias_ref, out_ref):
    kb = pl.program_id(1)
    kw = x_ref.shape[1]

    x_b = x_ref[...]
    ci = kb * kw + lax.broadcasted_iota(jnp.int32, x_b.shape, 1)
    sin = (jnp.where(ci == ii_ref[:, 0:1], vi_ref[:, 0:1], 0.0)
           + jnp.where(ci == ii_ref[:, 1:2], vi_ref[:, 1:2], 0.0))
    w_b = w_ref[...]
    p = jnp.dot(x_b, w_b, preferred_element_type=jnp.float32)
    bpart = jnp.dot(x_b * sin, w_b, preferred_element_type=jnp.float32)
    co = lax.broadcasted_iota(jnp.int32, p.shape, 1)
    sout = (jnp.where(co == io_ref[:, 0:1], vo_ref[:, 0:1], 0.0)
            + jnp.where(co == io_ref[:, 1:2], vo_ref[:, 1:2], 0.0))
    upd = p * sout + bpart

    @pl.when(kb == 0)
    def _():
        out_ref[...] = upd + bias_ref[...]

    @pl.when(kb != 0)
    def _():
        out_ref[...] += upd


def _run_sparse_dense(xf, w, vo, io, vi, ii, bias2d, tb, kbw):
    n, c1 = xf.shape
    c2 = w.shape[1]
    nk = c1 // kbw
    grid = (n // tb, nk)
    return pl.pallas_call(
        _sparse_dense_body,
        grid=grid,
        in_specs=[
            pl.BlockSpec((tb, kbw), lambda t, k: (t, k)),
            pl.BlockSpec((kbw, c2), lambda t, k: (k, 0)),
            pl.BlockSpec((tb, 2), lambda t, k: (t, 0)),
            pl.BlockSpec((tb, 2), lambda t, k: (t, 0)),
            pl.BlockSpec((tb, 2), lambda t, k: (t, 0)),
            pl.BlockSpec((tb, 2), lambda t, k: (t, 0)),
            pl.BlockSpec((1, c2), lambda t, k: (0, 0)),
        ],
        out_specs=pl.BlockSpec((tb, c2), lambda t, k: (t, 0)),
        out_shape=jax.ShapeDtypeStruct((n, c2), jnp.float32),
        compiler_params=pltpu.CompilerParams(
            dimension_semantics=("arbitrary", "arbitrary")),
    )(xf, w, vo, io, vi, ii, bias2d)


def kernel(x, weight, bias, so_cw, so_cb, so_bg, so_bb, so_bm, so_bv, so_ew,
           so_eb, si_cw, si_cb, si_bg, si_bb, si_bm, si_bv, si_ew, si_eb):
    b, l, c1 = x.shape
    c2 = weight.shape[1]
    n = b * l
    g = so_cw.shape[0]
    xf = x.reshape(n, c1)

    def gb(cw, cb, bg, bb, bm, bv, ew, eb):
        xr = xf.reshape(n, g, 4)
        comp = jnp.einsum('ngi,gi->ng', xr, cw) + cb
        bnv = (comp - bm) / jnp.sqrt(bv + 1e-5) * bg + bb
        act = jax.nn.gelu(bnv, approximate=False)
        return act @ ew.T + eb

    sc_o = jax.nn.softmax(
        gb(so_cw, so_cb, so_bg, so_bb, so_bm, so_bv, so_ew,
           so_eb).reshape(b, l, c2), axis=-1)
    sc_i = jax.nn.softmax(
        gb(si_cw, si_cb, si_bg, si_bb, si_bm, si_bv, si_ew,
           si_eb).reshape(b, l, c1), axis=-1)

    tn = min(256, n)
    vo, io = _run_top2(sc_o.reshape(n, c2), tn)
    vi, ii = _run_top2(sc_i.reshape(n, c1), tn)

    tb = min(512, n)
    kbw = min(512, c1)
    out = _run_sparse_dense(xf, weight, vo, io, vi, ii,
                            bias.reshape(1, c2), tb, kbw)
    return out.reshape(b, l, c2)


# speedup vs baseline: 1.0931x; 1.0274x over previous
import functools

import jax
import jax.numpy as jnp
from jax import lax
from jax.experimental import pallas as pl
from jax.experimental.pallas import tpu as pltpu


def _top2_body(s_ref, vals_ref, idxs_ref):
    s = s_ref[...]
    c = s.shape[-1]
    iota = lax.broadcasted_iota(jnp.int32, s.shape, 1)
    m1 = jnp.max(s, axis=-1, keepdims=True)
    i1 = jnp.min(jnp.where(s == m1, iota, c), axis=-1, keepdims=True)
    s2 = jnp.where(iota == i1, -jnp.inf, s)
    m2 = jnp.max(s2, axis=-1, keepdims=True)
    i2 = jnp.min(jnp.where(s2 == m2, iota, c), axis=-1, keepdims=True)
    vals_ref[:, 0:1] = m1
    vals_ref[:, 1:2] = m2
    idxs_ref[:, 0:1] = i1
    idxs_ref[:, 1:2] = i2


def _run_top2(scores, tn):
    n, c = scores.shape
    return pl.pallas_call(
        _top2_body,
        grid=(n // tn,),
        in_specs=[pl.BlockSpec((tn, c), lambda t: (t, 0))],
        out_specs=[
            pl.BlockSpec((tn, 2), lambda t: (t, 0)),
            pl.BlockSpec((tn, 2), lambda t: (t, 0)),
        ],
        out_shape=[
            jax.ShapeDtypeStruct((n, 2), jnp.float32),
            jax.ShapeDtypeStruct((n, 2), jnp.int32),
        ],
    )(scores)


def _sparse_dense_body(nk, x_ref, w_ref, vo_ref, io_ref, vi_ref, ii_ref,
                       bias_ref, out_ref, pacc, bacc):
    kb = pl.program_id(1)
    kw = x_ref.shape[1]

    @pl.when(kb == 0)
    def _():
        pacc[...] = jnp.zeros_like(pacc)
        bacc[...] = jnp.zeros_like(bacc)

    x_b = x_ref[...]
    ci = kb * kw + lax.broadcasted_iota(jnp.int32, x_b.shape, 1)
    sin = (jnp.where(ci == ii_ref[:, 0:1], vi_ref[:, 0:1], 0.0)
           + jnp.where(ci == ii_ref[:, 1:2], vi_ref[:, 1:2], 0.0))
    w_b = w_ref[...]
    pacc[...] += jnp.dot(x_b, w_b, preferred_element_type=jnp.float32)
    bacc[...] += jnp.dot(x_b * sin, w_b, preferred_element_type=jnp.float32)

    @pl.when(kb == nk - 1)
    def _():
        p = pacc[...]
        co = lax.broadcasted_iota(jnp.int32, p.shape, 1)
        sout = (jnp.where(co == io_ref[:, 0:1], vo_ref[:, 0:1], 0.0)
                + jnp.where(co == io_ref[:, 1:2], vo_ref[:, 1:2], 0.0))
        out_ref[...] = p * sout + bacc[...] + bias_ref[...]


def _run_sparse_dense(xf, w, vo, io, vi, ii, bias2d, tb, kbw):
    n, c1 = xf.shape
    c2 = w.shape[1]
    nk = c1 // kbw
    grid = (n // tb, nk)
    return pl.pallas_call(
        functools.partial(_sparse_dense_body, nk),
        grid=grid,
        in_specs=[
            pl.BlockSpec((tb, kbw), lambda t, k: (t, k)),
            pl.BlockSpec((kbw, c2), lambda t, k: (k, 0)),
            pl.BlockSpec((tb, 2), lambda t, k: (t, 0)),
            pl.BlockSpec((tb, 2), lambda t, k: (t, 0)),
            pl.BlockSpec((tb, 2), lambda t, k: (t, 0)),
            pl.BlockSpec((tb, 2), lambda t, k: (t, 0)),
            pl.BlockSpec((1, c2), lambda t, k: (0, 0)),
        ],
        out_specs=pl.BlockSpec((tb, c2), lambda t, k: (t, 0)),
        out_shape=jax.ShapeDtypeStruct((n, c2), jnp.float32),
        scratch_shapes=[
            pltpu.VMEM((tb, c2), jnp.float32),
            pltpu.VMEM((tb, c2), jnp.float32),
        ],
        compiler_params=pltpu.CompilerParams(
            dimension_semantics=("arbitrary", "arbitrary")),
    )(xf, w, vo, io, vi, ii, bias2d)


def kernel(x, weight, bias, so_cw, so_cb, so_bg, so_bb, so_bm, so_bv, so_ew,
           so_eb, si_cw, si_cb, si_bg, si_bb, si_bm, si_bv, si_ew, si_eb):
    b, l, c1 = x.shape
    c2 = weight.shape[1]
    n = b * l
    g = so_cw.shape[0]
    xf = x.reshape(n, c1)

    def gb(cw, cb, bg, bb, bm, bv, ew, eb):
        xr = xf.reshape(n, g, 4)
        comp = jnp.einsum('ngi,gi->ng', xr, cw) + cb
        bnv = (comp - bm) / jnp.sqrt(bv + 1e-5) * bg + bb
        act = jax.nn.gelu(bnv, approximate=False)
        return act @ ew.T + eb

    sc_o = jax.nn.softmax(
        gb(so_cw, so_cb, so_bg, so_bb, so_bm, so_bv, so_ew,
           so_eb).reshape(b, l, c2), axis=-1)
    sc_i = jax.nn.softmax(
        gb(si_cw, si_cb, si_bg, si_bb, si_bm, si_bv, si_ew,
           si_eb).reshape(b, l, c1), axis=-1)

    tn = min(256, n)
    vo, io = _run_top2(sc_o.reshape(n, c2), tn)
    vi, ii = _run_top2(sc_i.reshape(n, c1), tn)

    tb = min(512, n)
    kbw = min(512, c1)
    out = _run_sparse_dense(xf, weight, vo, io, vi, ii,
                            bias.reshape(1, c2), tb, kbw)
    return out.reshape(b, l, c2)
